# baseline (speedup 1.0000x reference)
import functools

import jax
import jax.numpy as jnp
from jax import lax
from jax.experimental import pallas as pl
from jax.experimental.pallas import tpu as pltpu
from jax.experimental.pallas import tpu_sc as plsc

B = 1024
V = 100000
TOP_N = 614
LANE = 128
CBLK = 2048
NBLK = (V + CBLK - 1) // CBLK

NC = 2
NS = 16
NW = NC * NS
BPW = B // NW


def _sc_gather_body(xflat_hbm, tgt_hbm, out_hbm, tgt_v, idx_v, val_v, sem):
    wid = lax.axis_index("s") * NC + lax.axis_index("c")
    base = wid * BPW
    pltpu.sync_copy(tgt_hbm.at[pl.ds(base, BPW)], tgt_v)
    for g in range(BPW // 16):
        t = tgt_v[pl.ds(g * 16, 16)]
        row = base + g * 16 + lax.iota(jnp.int32, 16)
        idx_v[pl.ds(g * 16, 16)] = row * V + t
    pltpu.async_copy(xflat_hbm.at[idx_v], val_v, sem).wait()
    pltpu.sync_copy(val_v, out_hbm.at[pl.ds(base, BPW)])


def _sc_gather(xflat, tgt):
    mesh = plsc.VectorSubcoreMesh(core_axis_name="c", subcore_axis_name="s")
    fn = functools.partial(
        pl.kernel,
        mesh=mesh,
        out_type=jax.ShapeDtypeStruct((B,), jnp.float32),
        scratch_types=[
            pltpu.VMEM((BPW,), jnp.int32),
            pltpu.VMEM((BPW,), jnp.int32),
            pltpu.VMEM((BPW,), jnp.float32),
            pltpu.SemaphoreType.DMA,
        ],
    )(_sc_gather_body)
    return fn(xflat, tgt)


def _lse_body(x_ref, lse_ref, m_ref, s_ref):
    pid = pl.program_id(0)

    @pl.when(pid == 0)
    def _init():
        m_ref[...] = jnp.full((B, LANE), -jnp.inf, dtype=jnp.float32)
        s_ref[...] = jnp.zeros((B, LANE), dtype=jnp.float32)

    def process(chunks):
        bm = chunks[0]
        for c in chunks[1:]:
            bm = jnp.maximum(bm, c)
        m_old = m_ref[...]
        m_new = jnp.maximum(m_old, bm)
        s = s_ref[...] * jnp.exp(m_old - m_new)
        for c in chunks:
            s = s + jnp.exp(c - m_new)
        m_ref[...] = m_new
        s_ref[...] = s

    @pl.when(pid < NBLK - 1)
    def _full():
        x = x_ref[...]
        process([x[:, k * LANE:(k + 1) * LANE] for k in range(CBLK // LANE)])

    @pl.when(pid == NBLK - 1)
    def _last():
        x = x_ref[...]
        lane = lax.broadcasted_iota(jnp.int32, (B, LANE), 1)
        base = pid * CBLK
        chunks = []
        for k in range(CBLK // LANE):
            col = base + k * LANE + lane
            chunks.append(jnp.where(col < V, x[:, k * LANE:(k + 1) * LANE],
                                    -jnp.inf))
        process(chunks)
        m = m_ref[...]
        s = s_ref[...]
        m_fin = jnp.max(m, axis=1, keepdims=True)
        s_fin = jnp.sum(s * jnp.exp(m - m_fin), axis=1, keepdims=True)
        lse_ref[...] = m_fin + jnp.log(s_fin)


def _tc_lse(x):
    return pl.pallas_call(
        _lse_body,
        grid=(NBLK,),
        in_specs=[pl.BlockSpec((B, CBLK), lambda i: (0, i))],
        out_specs=pl.BlockSpec((B, 1), lambda i: (0, 0)),
        out_shape=jax.ShapeDtypeStruct((B, 1), jnp.float32),
        scratch_shapes=[
            pltpu.VMEM((B, LANE), jnp.float32),
            pltpu.VMEM((B, LANE), jnp.float32),
        ],
    )(x)


def _topk_body(lse_ref, lseT_ref, xt_ref, xtT_ref, out_ref):
    loss_c = lse_ref[...] - xt_ref[...]
    loss_r = lseT_ref[...] - xtT_ref[...]
    gt = (loss_r > loss_c).astype(jnp.float32)
    eq = (loss_r == loss_c).astype(jnp.float32)
    c = jnp.sum(gt, axis=1, keepdims=True)
    e = jnp.sum(eq, axis=1, keepdims=True)
    w = jnp.clip(jnp.float32(TOP_N) - c, 0.0, e) / e
    out_ref[...] = jnp.sum(loss_c * w, keepdims=True) / jnp.float32(TOP_N)


def _tc_topk_mean(lse, xt):
    lse_t = jnp.reshape(lse, (1, B))
    xt_c = jnp.reshape(xt, (B, 1))
    xt_t = jnp.reshape(xt, (1, B))
    out = pl.pallas_call(
        _topk_body,
        out_shape=jax.ShapeDtypeStruct((1, 1), jnp.float32),
    )(lse, lse_t, xt_c, xt_t)
    return jnp.reshape(out, ())


def kernel(input, target):
    xflat = jnp.reshape(input, (B * V,))
    tgt = target.astype(jnp.int32)
    xt = _sc_gather(xflat, tgt)
    lse = _tc_lse(input)
    return _tc_topk_mean(lse, xt)
